# Initial kernel scaffold; baseline (speedup 1.0000x reference)
import jax
import jax.numpy as jnp
from jax.experimental import pallas as pl


def kernel(x, pos, edge_index, edge_type, batch, Wl, bl, Ws, bs, We, be):
    raise NotImplementedError("write your pallas kernel here")


# speedup vs baseline: 1.1851x; 1.1851x over previous
import functools

import jax
import jax.numpy as jnp
from jax.experimental import pallas as pl

N = 10000
E = 320000
D = 128
R = 7
L = 4
G8 = 8
BN = 400


def _dense_layer_body(g_ref, scl_ref, b_ref, c_ref, uv_ref, li_ref, an_ref,
                      wl_ref, ws_ref, h_ref, hn_ref):
    li = li_ref[...]
    acc = jnp.dot(li, ws_ref[...], preferred_element_type=jnp.float32)
    bmat = b_ref[...]
    for r in range(R):
        scl = scl_ref[r]
        cnt = scl[:, 0:1] + scl[:, 3:4]
        s1 = scl[:, 1:2] + scl[:, 4:5]
        s2 = scl[:, 2:3] + scl[:, 5:6]
        upd = (g_ref[r] + cnt * (bmat + c_ref[r:r + 1, :])
               + s1 * uv_ref[0:1, :] + s2 * uv_ref[1:2, :])
        acc += jnp.dot(upd, wl_ref[r], preferred_element_type=jnp.float32)
    h = jnp.maximum(acc + uv_ref[2:3, :], 0.0) + li
    h_ref[...] = h
    hn_ref[...] = h + an_ref[...]


def _dense_layer(g3, scl, bmat, cmat, uv, li, an, wl3, ws):
    nb = li.shape[0] // BN
    return pl.pallas_call(
        _dense_layer_body,
        grid=(nb,),
        in_specs=[
            pl.BlockSpec((R, BN, D), lambda i: (0, i, 0)),
            pl.BlockSpec((R, BN, 8), lambda i: (0, i, 0)),
            pl.BlockSpec((BN, D), lambda i: (i, 0)),
            pl.BlockSpec((R, D), lambda i: (0, 0)),
            pl.BlockSpec((8, D), lambda i: (0, 0)),
            pl.BlockSpec((BN, D), lambda i: (i, 0)),
            pl.BlockSpec((BN, D), lambda i: (i, 0)),
            pl.BlockSpec((R, D, D), lambda i: (0, 0, 0)),
            pl.BlockSpec((D, D), lambda i: (0, 0)),
        ],
        out_specs=[pl.BlockSpec((BN, D), lambda i: (i, 0)),
                   pl.BlockSpec((BN, D), lambda i: (i, 0))],
        out_shape=[jax.ShapeDtypeStruct((li.shape[0], D), jnp.float32)] * 2,
    )(g3, scl, bmat, cmat, uv, li, an, wl3, ws)


def _precompute_body(x_ref, w_ref, ab_ref, h0_ref):
    xb = x_ref[...]
    ab = jnp.dot(xb, w_ref[...], preferred_element_type=jnp.float32)
    ab_ref[...] = ab
    h0_ref[...] = xb + ab[:, :D]


def _precompute(x, wcat):
    nb = x.shape[0] // BN
    return pl.pallas_call(
        _precompute_body,
        grid=(nb,),
        in_specs=[pl.BlockSpec((BN, D), lambda i: (i, 0)),
                  pl.BlockSpec((D, 2 * L * D), lambda i: (0, 0))],
        out_specs=[pl.BlockSpec((BN, 2 * L * D), lambda i: (i, 0)),
                   pl.BlockSpec((BN, D), lambda i: (i, 0))],
        out_shape=[jax.ShapeDtypeStruct((x.shape[0], 2 * L * D), jnp.float32),
                   jax.ShapeDtypeStruct((x.shape[0], D), jnp.float32)],
    )(x, wcat)


def _pool_body(b_ref, xf_ref, out_ref):
    i = pl.program_id(0)
    oh = (b_ref[...] == jax.lax.broadcasted_iota(jnp.int32, (1, G8), 1))
    part = jax.lax.dot_general(oh.astype(jnp.float32), xf_ref[...],
                               (((0,), (0,)), ((), ())),
                               preferred_element_type=jnp.float32)

    @pl.when(i == 0)
    def _():
        out_ref[...] = part

    @pl.when(i > 0)
    def _():
        out_ref[...] += part


def _pool(batch2, xf):
    nb = xf.shape[0] // BN
    return pl.pallas_call(
        _pool_body,
        grid=(nb,),
        in_specs=[pl.BlockSpec((BN, 1), lambda i: (i, 0)),
                  pl.BlockSpec((BN, D), lambda i: (i, 0))],
        out_specs=pl.BlockSpec((G8, D), lambda i: (0, 0)),
        out_shape=jax.ShapeDtypeStruct((G8, D), jnp.float32),
    )(batch2, xf)


def kernel(x, pos, edge_index, edge_type, batch, Wl, bl, Ws, bs, We, be):
    src = edge_index[0]
    dst = edge_index[1]
    rel = edge_type[0]

    WeU = We[:, :D, :]
    WeV = We[:, D:2 * D, :]
    WeR = We[:, 2 * D:2 * D + R, :]
    u = We[:, 2 * D + R, :]
    v = We[:, 2 * D + R + 1, :]
    cmat = WeR + be[:, None, :]
    wcat = jnp.concatenate(
        [WeU[i] for i in range(L)] + [WeV[i] for i in range(L)], axis=1)
    uvb = jnp.concatenate(
        [u[:, None, :], v[:, None, :], (bl + bs)[:, None, :],
         jnp.zeros((L, 5, D), jnp.float32)], axis=1)
    wl3 = Wl.reshape(L, R, D, D)

    slot = rel * N + dst
    diff = pos[src] - pos[dst] + 1e-6
    dists = jnp.sqrt(jnp.sum(diff * diff, axis=1))
    seq = jnp.abs(src - dst).astype(jnp.float32)
    cnt = jax.ops.segment_sum(jnp.ones((E,), jnp.float32), slot,
                              num_segments=N * R).reshape(R, N)
    s1 = jax.ops.segment_sum(seq, slot, num_segments=N * R).reshape(R, N)
    s2 = jax.ops.segment_sum(dists, slot, num_segments=N * R).reshape(R, N)
    scl = jnp.concatenate(
        [cnt[..., None], s1[..., None], s2[..., None],
         jnp.zeros((R, N, 5), jnp.float32)], axis=-1)

    ab, h = _precompute(x, wcat)

    li = x
    zeros_nd = jnp.zeros((N, D), jnp.float32)
    for i in range(L):
        g3 = jax.ops.segment_sum(h[src], slot, num_segments=N * R
                                 ).reshape(R, N, D)
        an = ab[:, (i + 1) * D:(i + 2) * D] if i < L - 1 else zeros_nd
        bm = ab[:, (L + i) * D:(L + i + 1) * D]
        li, h = _dense_layer(g3, scl, bm, cmat[i], uvb[i], li, an, wl3[i],
                             Ws[i])

    graph_embedding = _pool(batch[:, None].astype(jnp.int32), li)
    return li, graph_embedding
